# Initial kernel scaffold; baseline (speedup 1.0000x reference)
import jax
import jax.numpy as jnp
from jax.experimental import pallas as pl


def kernel(encoded_sources, sources, targets, emb, Ws_w, Ws_b, Wc_w, Wc_b, Wo_w, Wo_b, W_ih, b_ih, W_hh, b_hh):
    raise NotImplementedError("write your pallas kernel here")


# speedup vs baseline: 2.1113x; 2.1113x over previous
import jax
import jax.numpy as jnp
from jax import lax
from jax.experimental import pallas as pl
from jax.experimental.pallas import tpu as pltpu
from jax.experimental.pallas import tpu_sc as plsc

B, SEQ, T = 1024, 200, 8
H, E, V, OOV = 128, 64, 10000, 50
UNK = 1
VO = V + OOV
SP = 208
NC, NS, LANES = 2, 16, 16
NW = NC * NS
EP = 128
BT = 32
NB = B // BT
ROWS_PW = B // NW
IDX_PW = (B * T) // NW


def _emb_gather_body(emb_hbm, idx_hbm, out_hbm, idxbuf, rows, sem):
    cid = lax.axis_index("c")
    sid = lax.axis_index("s")
    wid = sid * NC + cid
    pltpu.sync_copy(idx_hbm.at[wid], idxbuf)
    for j in range(IDX_PW // 128):
        pltpu.async_copy(emb_hbm.at[idxbuf.at[j]],
                         rows.at[pl.ds(j * 128, 128)], sem).wait()
    pltpu.sync_copy(rows, out_hbm.at[pl.ds(wid * IDX_PW, IDX_PW)])


def _emb_gather(emb, idx3):
    mesh = plsc.VectorSubcoreMesh(core_axis_name="c", subcore_axis_name="s",
                                  num_cores=NC, num_subcores=NS)
    fn = pl.kernel(
        _emb_gather_body,
        out_type=jax.ShapeDtypeStruct((B * T, EP), jnp.float32),
        mesh=mesh,
        compiler_params=pltpu.CompilerParams(needs_layout_passes=False),
        scratch_types=[
            pltpu.VMEM((IDX_PW // 128, 128), jnp.int32),
            pltpu.VMEM((IDX_PW, EP), jnp.float32),
            pltpu.SemaphoreType.DMA,
        ],
    )
    return fn(emb, idx3)


def _tc_body(src_ref, tgt_ref, temb_ref, enc_ref,
             wst, wsb, wct, wcb, wot, wob, wih, bih, whh, bhh,
             out_ref, pc_ref, state_ref, weighted_ref, wc_ref):
    i = pl.program_id(1)
    src = src_ref[...][:, :SEQ]
    enc = enc_ref[...]

    @pl.when(i == 0)
    def _init():
        slen = jnp.sum((src > 0).astype(jnp.int32), axis=1, keepdims=True)
        last_idx = jnp.clip(slen - 1, 0, SEQ - 1)
        seq_iota = lax.broadcasted_iota(jnp.int32, (BT, SEQ), 1)
        sel = (seq_iota == last_idx).astype(jnp.float32)
        last_step = lax.dot_general(
            sel, enc, (((1,), (1,)), ((0,), (0,))),
            preferred_element_type=jnp.float32)
        state_ref[...] = jnp.dot(last_step, wst[...],
                                 preferred_element_type=jnp.float32) + wsb[...]
        weighted_ref[...] = jnp.zeros((BT, 2 * H), jnp.float32)
        wc_ref[...] = jnp.tanh(
            jnp.dot(enc.reshape(BT * SEQ, 2 * H), wct[...],
                    preferred_element_type=jnp.float32) + wcb[...]
        ).reshape(BT, SEQ, H)

    state = state_ref[...]
    wc = wc_ref[...]

    x = jnp.concatenate([temb_ref[0, :, :E], weighted_ref[...]], axis=1)
    gi = jnp.dot(x, wih[...], preferred_element_type=jnp.float32) + bih[...]
    gh = jnp.dot(state, whh[...], preferred_element_type=jnp.float32) + bhh[...]
    r = 1.0 / (1.0 + jnp.exp(-(gi[:, :H] + gh[:, :H])))
    z = 1.0 / (1.0 + jnp.exp(-(gi[:, H:2 * H] + gh[:, H:2 * H])))
    n = jnp.tanh(gi[:, 2 * H:] + r * gh[:, 2 * H:])
    state = (1.0 - z) * n + z * state
    state_ref[...] = state

    score_g = jnp.dot(state, wot[...],
                      preferred_element_type=jnp.float32) + wob[...]
    enc_mask = jnp.where(src == 0, -1000.0, 0.0)
    score_c = jnp.tanh(lax.dot_general(
        wc, state, (((2,), (1,)), ((0,), (0,))),
        preferred_element_type=jnp.float32)) + enc_mask

    m = jnp.maximum(jnp.max(score_g, axis=1, keepdims=True),
                    jnp.max(score_c, axis=1, keepdims=True))
    eg = jnp.exp(score_g - m)
    ec = jnp.exp(score_c - m)
    inv = 1.0 / (jnp.sum(eg, axis=1, keepdims=True)
                 + jnp.sum(ec, axis=1, keepdims=True))
    oov_fill = jnp.full((BT, OOV), 1e-5, jnp.float32)
    out_ref[0] = jnp.concatenate([eg * inv, oov_fill], axis=1)
    pc = ec * inv
    pc_ref[0] = jnp.concatenate(
        [pc, jnp.zeros((BT, SP - SEQ), jnp.float32)], axis=1)

    mask2 = (src == tgt_ref[0]).astype(jnp.float32)
    ssum = jnp.sum(mask2, axis=1, keepdims=True)
    idxn = mask2 / jnp.maximum(ssum, 1.0)
    weighted_ref[...] = lax.dot_general(
        pc * idxn, enc, (((1,), (1,)), ((0,), (0,))),
        preferred_element_type=jnp.float32)


def _tc_forward(src_pad, targets, temb_t, enc,
                wst, wsb, wct, wcb, wot, wob, wih, bih, whh, bhh):
    f32 = jnp.float32
    in_specs = [
        pl.BlockSpec((BT, SP), lambda g, i: (g, 0)),
        pl.BlockSpec((1, BT, 1), lambda g, i: (i, g, 0)),
        pl.BlockSpec((1, BT, EP), lambda g, i: (i, g, 0)),
        pl.BlockSpec((BT, SEQ, 2 * H), lambda g, i: (g, 0, 0)),
        pl.BlockSpec((2 * H, H), lambda g, i: (0, 0)),
        pl.BlockSpec((1, H), lambda g, i: (0, 0)),
        pl.BlockSpec((2 * H, H), lambda g, i: (0, 0)),
        pl.BlockSpec((1, H), lambda g, i: (0, 0)),
        pl.BlockSpec((H, V), lambda g, i: (0, 0)),
        pl.BlockSpec((1, V), lambda g, i: (0, 0)),
        pl.BlockSpec((E + 2 * H, 3 * H), lambda g, i: (0, 0)),
        pl.BlockSpec((1, 3 * H), lambda g, i: (0, 0)),
        pl.BlockSpec((H, 3 * H), lambda g, i: (0, 0)),
        pl.BlockSpec((1, 3 * H), lambda g, i: (0, 0)),
    ]
    out_specs = [
        pl.BlockSpec((1, BT, VO), lambda g, i: (i, g, 0)),
        pl.BlockSpec((1, BT, SP), lambda g, i: (i, g, 0)),
    ]
    out_shape = [
        jax.ShapeDtypeStruct((T, B, VO), f32),
        jax.ShapeDtypeStruct((T, B, SP), f32),
    ]
    return pl.pallas_call(
        _tc_body,
        grid=(NB, T),
        in_specs=in_specs,
        out_specs=out_specs,
        out_shape=out_shape,
        scratch_shapes=[
            pltpu.VMEM((BT, H), f32),
            pltpu.VMEM((BT, 2 * H), f32),
            pltpu.VMEM((BT, SEQ, H), f32),
        ],
        compiler_params=pltpu.CompilerParams(
            dimension_semantics=("arbitrary", "arbitrary"),
            vmem_limit_bytes=100 * 1024 * 1024),
    )(src_pad, targets, temb_t, enc,
      wst, wsb, wct, wcb, wot, wob, wih, bih, whh, bhh)


def _scatter_body(base_hbm, src_hbm, pc_hbm, out_hbm, srcbuf, pcbuf, rowbuf):
    cid = lax.axis_index("c")
    sid = lax.axis_index("s")
    wid = sid * NC + cid

    def row_step(t, carry):
        b = wid * ROWS_PW + t
        pltpu.sync_copy(src_hbm.at[b], srcbuf)
        for i in range(T):
            pltpu.sync_copy(pc_hbm.at[i, b], pcbuf.at[i])
            pltpu.sync_copy(base_hbm.at[i, b], rowbuf.at[i])
        for i in range(T):
            ivec = jnp.full((LANES,), i, jnp.int32)
            for ch in range(SP // LANES):
                sidx = srcbuf[pl.ds(ch * LANES, LANES)]
                vals = pcbuf[i, pl.ds(ch * LANES, LANES)]
                plsc.addupdate_scatter(rowbuf, [ivec, sidx], vals)
        pltpu.sync_copy(rowbuf, out_hbm.at[b])
        return carry

    lax.fori_loop(0, ROWS_PW, row_step, 0)


def _sc_scatter(base_t, src_pad, pc_t):
    mesh = plsc.VectorSubcoreMesh(core_axis_name="c", subcore_axis_name="s",
                                  num_cores=NC, num_subcores=NS)
    fn = pl.kernel(
        _scatter_body,
        out_type=jax.ShapeDtypeStruct((B, T, VO), jnp.float32),
        mesh=mesh,
        compiler_params=pltpu.CompilerParams(needs_layout_passes=False),
        scratch_types=[
            pltpu.VMEM((SP,), jnp.int32),
            pltpu.VMEM((T, SP), jnp.float32),
            pltpu.VMEM((T, VO), jnp.float32),
        ],
    )
    return fn(base_t, src_pad, pc_t)


def kernel(encoded_sources, sources, targets, emb,
           Ws_w, Ws_b, Wc_w, Wc_b, Wo_w, Wo_b, W_ih, b_ih, W_hh, b_hh):
    f32 = jnp.float32
    sources = sources.astype(jnp.int32)
    targets = targets.astype(jnp.int32)
    src_pad = jnp.pad(sources, ((0, 0), (0, SP - SEQ)))

    tgt_idx = jnp.where(targets >= V, UNK, targets).T.reshape(NW, IDX_PW // 128, 128)
    emb_pad = jnp.pad(emb.astype(f32), ((0, 0), (0, EP - E)))
    temb_t = _emb_gather(emb_pad, tgt_idx).reshape(T, B, EP)

    out_base, pc = _tc_forward(
        src_pad, targets.T.reshape(T, B, 1), temb_t, encoded_sources,
        Ws_w.T, Ws_b.reshape(1, H), Wc_w.T, Wc_b.reshape(1, H),
        Wo_w.T, Wo_b.reshape(1, V), W_ih.T, b_ih.reshape(1, 3 * H),
        W_hh.T, b_hh.reshape(1, 3 * H))

    return _sc_scatter(out_base, src_pad, pc)


# speedup vs baseline: 2.3995x; 1.1365x over previous
import jax
import jax.numpy as jnp
from jax import lax
from jax.experimental import pallas as pl
from jax.experimental.pallas import tpu as pltpu
from jax.experimental.pallas import tpu_sc as plsc

B, SEQ, T = 1024, 200, 8
H, E, V, OOV = 128, 64, 10000, 50
UNK = 1
VO = V + OOV
SP = 208
NC, NS, LANES = 2, 16, 16
NW = NC * NS
EP = 128
BT = 32
NB = B // BT
ROWS_PW = B // NW
IDX_PW = (B * T) // NW


def _emb_gather_body(emb_hbm, idx_hbm, out_hbm, idxbuf, rows, sem):
    cid = lax.axis_index("c")
    sid = lax.axis_index("s")
    wid = sid * NC + cid
    pltpu.sync_copy(idx_hbm.at[wid], idxbuf)
    for j in range(IDX_PW // 128):
        pltpu.async_copy(emb_hbm.at[idxbuf.at[j]],
                         rows.at[pl.ds(j * 128, 128)], sem).wait()
    pltpu.sync_copy(rows, out_hbm.at[pl.ds(wid * IDX_PW, IDX_PW)])


def _emb_gather(emb, idx3):
    mesh = plsc.VectorSubcoreMesh(core_axis_name="c", subcore_axis_name="s",
                                  num_cores=NC, num_subcores=NS)
    fn = pl.kernel(
        _emb_gather_body,
        out_type=jax.ShapeDtypeStruct((B * T, EP), jnp.float32),
        mesh=mesh,
        compiler_params=pltpu.CompilerParams(needs_layout_passes=False),
        scratch_types=[
            pltpu.VMEM((IDX_PW // 128, 128), jnp.int32),
            pltpu.VMEM((IDX_PW, EP), jnp.float32),
            pltpu.SemaphoreType.DMA,
        ],
    )
    return fn(emb, idx3)


def _tc_body(src_ref, tgt_ref, temb_ref, enc_ref,
             wst, wsb, wct, wcb, wot, wob, wih, bih, whh, bhh,
             out_ref, pc_ref, state_ref, weighted_ref, wc_ref):
    i = pl.program_id(1)
    src = src_ref[...][:, :SEQ]
    enc = enc_ref[...]

    @pl.when(i == 0)
    def _init():
        slen = jnp.sum((src > 0).astype(jnp.int32), axis=1, keepdims=True)
        last_idx = jnp.clip(slen - 1, 0, SEQ - 1)
        seq_iota = lax.broadcasted_iota(jnp.int32, (BT, SEQ), 1)
        sel = (seq_iota == last_idx).astype(jnp.float32)
        last_step = lax.dot_general(
            sel, enc, (((1,), (1,)), ((0,), (0,))),
            preferred_element_type=jnp.float32)
        state_ref[...] = jnp.dot(last_step, wst[...],
                                 preferred_element_type=jnp.float32) + wsb[...]
        weighted_ref[...] = jnp.zeros((BT, 2 * H), jnp.float32)
        wc_ref[...] = jnp.tanh(
            jnp.dot(enc.reshape(BT * SEQ, 2 * H), wct[...],
                    preferred_element_type=jnp.float32) + wcb[...]
        ).reshape(BT, SEQ, H)

    state = state_ref[...]
    wc = wc_ref[...]

    x = jnp.concatenate([temb_ref[0, :, :E], weighted_ref[...]], axis=1)
    gi = jnp.dot(x, wih[...], preferred_element_type=jnp.float32) + bih[...]
    gh = jnp.dot(state, whh[...], preferred_element_type=jnp.float32) + bhh[...]
    r = 1.0 / (1.0 + jnp.exp(-(gi[:, :H] + gh[:, :H])))
    z = 1.0 / (1.0 + jnp.exp(-(gi[:, H:2 * H] + gh[:, H:2 * H])))
    n = jnp.tanh(gi[:, 2 * H:] + r * gh[:, 2 * H:])
    state = (1.0 - z) * n + z * state
    state_ref[...] = state

    score_g = jnp.dot(state, wot[...],
                      preferred_element_type=jnp.float32) + wob[...]
    enc_mask = jnp.where(src == 0, -1000.0, 0.0)
    score_c = jnp.tanh(lax.dot_general(
        wc, state, (((2,), (1,)), ((0,), (0,))),
        preferred_element_type=jnp.float32)) + enc_mask

    m = jnp.maximum(jnp.max(score_g, axis=1, keepdims=True),
                    jnp.max(score_c, axis=1, keepdims=True))
    eg = jnp.exp(score_g - m)
    ec = jnp.exp(score_c - m)
    inv = 1.0 / (jnp.sum(eg, axis=1, keepdims=True)
                 + jnp.sum(ec, axis=1, keepdims=True))
    oov_fill = jnp.full((BT, OOV), 1e-5, jnp.float32)
    out_ref[0] = jnp.concatenate([eg * inv, oov_fill], axis=1)
    pc = ec * inv
    pc_ref[0] = jnp.concatenate(
        [pc, jnp.zeros((BT, SP - SEQ), jnp.float32)], axis=1)

    mask2 = (src == tgt_ref[0]).astype(jnp.float32)
    ssum = jnp.sum(mask2, axis=1, keepdims=True)
    idxn = mask2 / jnp.maximum(ssum, 1.0)
    weighted_ref[...] = lax.dot_general(
        pc * idxn, enc, (((1,), (1,)), ((0,), (0,))),
        preferred_element_type=jnp.float32)


def _tc_forward(src_pad, targets, temb_t, enc,
                wst, wsb, wct, wcb, wot, wob, wih, bih, whh, bhh):
    f32 = jnp.float32
    in_specs = [
        pl.BlockSpec((BT, SP), lambda g, i: (g, 0)),
        pl.BlockSpec((1, BT, 1), lambda g, i: (i, g, 0)),
        pl.BlockSpec((1, BT, EP), lambda g, i: (i, g, 0)),
        pl.BlockSpec((BT, SEQ, 2 * H), lambda g, i: (g, 0, 0)),
        pl.BlockSpec((2 * H, H), lambda g, i: (0, 0)),
        pl.BlockSpec((1, H), lambda g, i: (0, 0)),
        pl.BlockSpec((2 * H, H), lambda g, i: (0, 0)),
        pl.BlockSpec((1, H), lambda g, i: (0, 0)),
        pl.BlockSpec((H, V), lambda g, i: (0, 0)),
        pl.BlockSpec((1, V), lambda g, i: (0, 0)),
        pl.BlockSpec((E + 2 * H, 3 * H), lambda g, i: (0, 0)),
        pl.BlockSpec((1, 3 * H), lambda g, i: (0, 0)),
        pl.BlockSpec((H, 3 * H), lambda g, i: (0, 0)),
        pl.BlockSpec((1, 3 * H), lambda g, i: (0, 0)),
    ]
    out_specs = [
        pl.BlockSpec((1, BT, VO), lambda g, i: (i, g, 0)),
        pl.BlockSpec((1, BT, SP), lambda g, i: (i, g, 0)),
    ]
    out_shape = [
        jax.ShapeDtypeStruct((T, B, VO), f32),
        jax.ShapeDtypeStruct((T, B, SP), f32),
    ]
    return pl.pallas_call(
        _tc_body,
        grid=(NB, T),
        in_specs=in_specs,
        out_specs=out_specs,
        out_shape=out_shape,
        scratch_shapes=[
            pltpu.VMEM((BT, H), f32),
            pltpu.VMEM((BT, 2 * H), f32),
            pltpu.VMEM((BT, SEQ, H), f32),
        ],
        compiler_params=pltpu.CompilerParams(
            dimension_semantics=("arbitrary", "arbitrary"),
            vmem_limit_bytes=100 * 1024 * 1024),
    )(src_pad, targets, temb_t, enc,
      wst, wsb, wct, wcb, wot, wob, wih, bih, whh, bhh)


HT = T // 2


def _scatter_body(base_hbm, src_hbm, pc_hbm, out_hbm,
                  srcbuf, pcbuf, rowbuf, sem0, sem1):
    cid = lax.axis_index("c")
    sid = lax.axis_index("s")
    wid = sid * NC + cid
    b0 = wid * ROWS_PW
    sems = (sem0, sem1)

    def issue_in(b, k):
        pltpu.async_copy(src_hbm.at[b], srcbuf.at[k], sems[k])
        pltpu.async_copy(pc_hbm.at[pl.ds(k * HT, HT), b], pcbuf.at[k], sems[k])
        pltpu.async_copy(base_hbm.at[pl.ds(k * HT, HT), b], rowbuf.at[k], sems[k])

    def wait_in(b, k):
        pltpu.make_async_copy(src_hbm.at[b], srcbuf.at[k], sems[k]).wait()
        pltpu.make_async_copy(pc_hbm.at[pl.ds(k * HT, HT), b], pcbuf.at[k], sems[k]).wait()
        pltpu.make_async_copy(base_hbm.at[pl.ds(k * HT, HT), b], rowbuf.at[k], sems[k]).wait()

    issue_in(b0, 0)

    def outer(o, carry):
        b = b0 + o
        for k in range(2):
            wait_in(b, k)

            if k == 0:
                issue_in(b, 1)
            else:
                @pl.when(o + 1 < ROWS_PW)
                def _prefetch_next_row():
                    issue_in(b + 1, 0)

            for i in range(HT):
                ivec = jnp.full((LANES,), i, jnp.int32)
                for ch in range(SP // LANES):
                    sidx = srcbuf[k, pl.ds(ch * LANES, LANES)]
                    vals = pcbuf[k, i, pl.ds(ch * LANES, LANES)]
                    plsc.addupdate_scatter(rowbuf.at[k], [ivec, sidx], vals)
            pltpu.sync_copy(rowbuf.at[k],
                            out_hbm.at[b, pl.ds(k * HT, HT)])
        return carry

    lax.fori_loop(0, ROWS_PW, outer, 0)


def _sc_scatter(base_t, src_pad, pc_t):
    mesh = plsc.VectorSubcoreMesh(core_axis_name="c", subcore_axis_name="s",
                                  num_cores=NC, num_subcores=NS)
    fn = pl.kernel(
        _scatter_body,
        out_type=jax.ShapeDtypeStruct((B, T, VO), jnp.float32),
        mesh=mesh,
        compiler_params=pltpu.CompilerParams(needs_layout_passes=False),
        scratch_types=[
            pltpu.VMEM((2, SP), jnp.int32),
            pltpu.VMEM((2, HT, SP), jnp.float32),
            pltpu.VMEM((2, HT, VO), jnp.float32),
            pltpu.SemaphoreType.DMA,
            pltpu.SemaphoreType.DMA,
        ],
    )
    return fn(base_t, src_pad, pc_t)


def kernel(encoded_sources, sources, targets, emb,
           Ws_w, Ws_b, Wc_w, Wc_b, Wo_w, Wo_b, W_ih, b_ih, W_hh, b_hh):
    f32 = jnp.float32
    sources = sources.astype(jnp.int32)
    targets = targets.astype(jnp.int32)
    src_pad = jnp.pad(sources, ((0, 0), (0, SP - SEQ)))

    tgt_idx = jnp.where(targets >= V, UNK, targets).T.reshape(NW, IDX_PW // 128, 128)
    emb_pad = jnp.pad(emb.astype(f32), ((0, 0), (0, EP - E)))
    temb_t = _emb_gather(emb_pad, tgt_idx).reshape(T, B, EP)

    out_base, pc = _tc_forward(
        src_pad, targets.T.reshape(T, B, 1), temb_t, encoded_sources,
        Ws_w.T, Ws_b.reshape(1, H), Wc_w.T, Wc_b.reshape(1, H),
        Wo_w.T, Wo_b.reshape(1, V), W_ih.T, b_ih.reshape(1, 3 * H),
        W_hh.T, b_hh.reshape(1, 3 * H))

    return _sc_scatter(out_base, src_pad, pc)


# speedup vs baseline: 2.4784x; 1.0329x over previous
import jax
import jax.numpy as jnp
from jax import lax
from jax.experimental import pallas as pl
from jax.experimental.pallas import tpu as pltpu
from jax.experimental.pallas import tpu_sc as plsc

B, SEQ, T = 1024, 200, 8
H, E, V, OOV = 128, 64, 10000, 50
UNK = 1
VO = V + OOV
SP = 208
NC, NS, LANES = 2, 16, 16
NW = NC * NS
EP = 128
BT = 64
NB = B // BT
ROWS_PW = B // NW
IDX_PW = (B * T) // NW


def _emb_gather_body(emb_hbm, idx_hbm, out_hbm, idxbuf, rows, sem):
    cid = lax.axis_index("c")
    sid = lax.axis_index("s")
    wid = sid * NC + cid
    pltpu.sync_copy(idx_hbm.at[wid], idxbuf)
    for j in range(IDX_PW // 128):
        pltpu.async_copy(emb_hbm.at[idxbuf.at[j]],
                         rows.at[pl.ds(j * 128, 128)], sem).wait()
    pltpu.sync_copy(rows, out_hbm.at[pl.ds(wid * IDX_PW, IDX_PW)])


def _emb_gather(emb, idx3):
    mesh = plsc.VectorSubcoreMesh(core_axis_name="c", subcore_axis_name="s",
                                  num_cores=NC, num_subcores=NS)
    fn = pl.kernel(
        _emb_gather_body,
        out_type=jax.ShapeDtypeStruct((B * T, EP), jnp.float32),
        mesh=mesh,
        compiler_params=pltpu.CompilerParams(needs_layout_passes=False),
        scratch_types=[
            pltpu.VMEM((IDX_PW // 128, 128), jnp.int32),
            pltpu.VMEM((IDX_PW, EP), jnp.float32),
            pltpu.SemaphoreType.DMA,
        ],
    )
    return fn(emb, idx3)


def _tc_body(src_ref, tgt_ref, temb_ref, enc_hbm,
             wst, wsb, wct, wcb, wot, wob, wih, bih, whh, bhh,
             out_ref, pc_ref, state_ref, weighted_ref, wc_ref, enc_vmem,
             enc_sem):
    g = pl.program_id(0)
    i = pl.program_id(1)
    src = src_ref[...][:, :SEQ]

    @pl.when(i == 0)
    def _stage_enc():
        pltpu.async_copy(enc_hbm.at[pl.ds(g * BT, BT)], enc_vmem,
                         enc_sem).wait()

    enc = enc_vmem[...]

    @pl.when(i == 0)
    def _init():
        slen = jnp.sum((src > 0).astype(jnp.int32), axis=1, keepdims=True)
        last_idx = jnp.clip(slen - 1, 0, SEQ - 1)
        seq_iota = lax.broadcasted_iota(jnp.int32, (BT, SEQ), 1)
        sel = (seq_iota == last_idx).astype(jnp.float32)
        last_step = lax.dot_general(
            sel, enc, (((1,), (1,)), ((0,), (0,))),
            preferred_element_type=jnp.float32)
        state_ref[...] = jnp.dot(last_step, wst[...],
                                 preferred_element_type=jnp.float32) + wsb[...]
        weighted_ref[...] = jnp.zeros((BT, 2 * H), jnp.float32)
        wc_ref[...] = jnp.tanh(
            jnp.dot(enc.reshape(BT * SEQ, 2 * H), wct[...],
                    preferred_element_type=jnp.float32) + wcb[...]
        ).reshape(BT, SEQ, H)

    state = state_ref[...]
    wc = wc_ref[...]

    x = jnp.concatenate([temb_ref[0, :, :E], weighted_ref[...]], axis=1)
    gi = jnp.dot(x, wih[...], preferred_element_type=jnp.float32) + bih[...]
    gh = jnp.dot(state, whh[...], preferred_element_type=jnp.float32) + bhh[...]
    r = 1.0 / (1.0 + jnp.exp(-(gi[:, :H] + gh[:, :H])))
    z = 1.0 / (1.0 + jnp.exp(-(gi[:, H:2 * H] + gh[:, H:2 * H])))
    n = jnp.tanh(gi[:, 2 * H:] + r * gh[:, 2 * H:])
    state = (1.0 - z) * n + z * state
    state_ref[...] = state

    score_g = jnp.dot(state, wot[...],
                      preferred_element_type=jnp.float32) + wob[...]
    enc_mask = jnp.where(src == 0, -1000.0, 0.0)
    score_c = jnp.tanh(lax.dot_general(
        wc, state, (((2,), (1,)), ((0,), (0,))),
        preferred_element_type=jnp.float32)) + enc_mask

    m = jnp.maximum(jnp.max(score_g, axis=1, keepdims=True),
                    jnp.max(score_c, axis=1, keepdims=True))
    eg = jnp.exp(score_g - m)
    ec = jnp.exp(score_c - m)
    inv = 1.0 / (jnp.sum(eg, axis=1, keepdims=True)
                 + jnp.sum(ec, axis=1, keepdims=True))
    oov_fill = jnp.full((BT, OOV), 1e-5, jnp.float32)
    out_ref[0] = jnp.concatenate([eg * inv, oov_fill], axis=1)
    pc = ec * inv
    pc_ref[0] = jnp.concatenate(
        [pc, jnp.zeros((BT, SP - SEQ), jnp.float32)], axis=1)

    mask2 = (src == tgt_ref[0]).astype(jnp.float32)
    ssum = jnp.sum(mask2, axis=1, keepdims=True)
    idxn = mask2 / jnp.maximum(ssum, 1.0)
    weighted_ref[...] = lax.dot_general(
        pc * idxn, enc, (((1,), (1,)), ((0,), (0,))),
        preferred_element_type=jnp.float32)


def _tc_forward(src_pad, targets, temb_t, enc,
                wst, wsb, wct, wcb, wot, wob, wih, bih, whh, bhh):
    f32 = jnp.float32
    in_specs = [
        pl.BlockSpec((BT, SP), lambda g, i: (g, 0)),
        pl.BlockSpec((1, BT, 1), lambda g, i: (i, g, 0)),
        pl.BlockSpec((1, BT, EP), lambda g, i: (i, g, 0)),
        pl.BlockSpec(memory_space=pl.ANY),
        pl.BlockSpec((2 * H, H), lambda g, i: (0, 0)),
        pl.BlockSpec((1, H), lambda g, i: (0, 0)),
        pl.BlockSpec((2 * H, H), lambda g, i: (0, 0)),
        pl.BlockSpec((1, H), lambda g, i: (0, 0)),
        pl.BlockSpec((H, V), lambda g, i: (0, 0)),
        pl.BlockSpec((1, V), lambda g, i: (0, 0)),
        pl.BlockSpec((E + 2 * H, 3 * H), lambda g, i: (0, 0)),
        pl.BlockSpec((1, 3 * H), lambda g, i: (0, 0)),
        pl.BlockSpec((H, 3 * H), lambda g, i: (0, 0)),
        pl.BlockSpec((1, 3 * H), lambda g, i: (0, 0)),
    ]
    out_specs = [
        pl.BlockSpec((1, BT, VO), lambda g, i: (i, g, 0)),
        pl.BlockSpec((1, BT, SP), lambda g, i: (i, g, 0)),
    ]
    out_shape = [
        jax.ShapeDtypeStruct((T, B, VO), f32),
        jax.ShapeDtypeStruct((T, B, SP), f32),
    ]
    return pl.pallas_call(
        _tc_body,
        grid=(NB, T),
        in_specs=in_specs,
        out_specs=out_specs,
        out_shape=out_shape,
        scratch_shapes=[
            pltpu.VMEM((BT, H), f32),
            pltpu.VMEM((BT, 2 * H), f32),
            pltpu.VMEM((BT, SEQ, H), f32),
            pltpu.VMEM((BT, SEQ, 2 * H), f32),
            pltpu.SemaphoreType.DMA,
        ],
        compiler_params=pltpu.CompilerParams(
            dimension_semantics=("arbitrary", "arbitrary"),
            vmem_limit_bytes=100 * 1024 * 1024),
    )(src_pad, targets, temb_t, enc,
      wst, wsb, wct, wcb, wot, wob, wih, bih, whh, bhh)


HT = T // 2


def _scatter_body(base_hbm, src_hbm, pc_hbm, out_hbm,
                  srcbuf, pcbuf, rowbuf, sem0, sem1):
    cid = lax.axis_index("c")
    sid = lax.axis_index("s")
    wid = sid * NC + cid
    b0 = wid * ROWS_PW
    sems = (sem0, sem1)

    def issue_in(b, k):
        pltpu.async_copy(src_hbm.at[b], srcbuf.at[k], sems[k])
        pltpu.async_copy(pc_hbm.at[pl.ds(k * HT, HT), b], pcbuf.at[k], sems[k])
        pltpu.async_copy(base_hbm.at[pl.ds(k * HT, HT), b], rowbuf.at[k], sems[k])

    def wait_in(b, k):
        pltpu.make_async_copy(src_hbm.at[b], srcbuf.at[k], sems[k]).wait()
        pltpu.make_async_copy(pc_hbm.at[pl.ds(k * HT, HT), b], pcbuf.at[k], sems[k]).wait()
        pltpu.make_async_copy(base_hbm.at[pl.ds(k * HT, HT), b], rowbuf.at[k], sems[k]).wait()

    issue_in(b0, 0)

    def outer(o, carry):
        b = b0 + o
        for k in range(2):
            wait_in(b, k)

            if k == 0:
                issue_in(b, 1)
            else:
                @pl.when(o + 1 < ROWS_PW)
                def _prefetch_next_row():
                    issue_in(b + 1, 0)

            for i in range(HT):
                ivec = jnp.full((LANES,), i, jnp.int32)
                for ch in range(SP // LANES):
                    sidx = srcbuf[k, pl.ds(ch * LANES, LANES)]
                    vals = pcbuf[k, i, pl.ds(ch * LANES, LANES)]
                    plsc.addupdate_scatter(rowbuf.at[k], [ivec, sidx], vals)
            pltpu.sync_copy(rowbuf.at[k],
                            out_hbm.at[b, pl.ds(k * HT, HT)])
        return carry

    lax.fori_loop(0, ROWS_PW, outer, 0)


def _sc_scatter(base_t, src_pad, pc_t):
    mesh = plsc.VectorSubcoreMesh(core_axis_name="c", subcore_axis_name="s",
                                  num_cores=NC, num_subcores=NS)
    fn = pl.kernel(
        _scatter_body,
        out_type=jax.ShapeDtypeStruct((B, T, VO), jnp.float32),
        mesh=mesh,
        compiler_params=pltpu.CompilerParams(needs_layout_passes=False),
        scratch_types=[
            pltpu.VMEM((2, SP), jnp.int32),
            pltpu.VMEM((2, HT, SP), jnp.float32),
            pltpu.VMEM((2, HT, VO), jnp.float32),
            pltpu.SemaphoreType.DMA,
            pltpu.SemaphoreType.DMA,
        ],
    )
    return fn(base_t, src_pad, pc_t)


def kernel(encoded_sources, sources, targets, emb,
           Ws_w, Ws_b, Wc_w, Wc_b, Wo_w, Wo_b, W_ih, b_ih, W_hh, b_hh):
    f32 = jnp.float32
    sources = sources.astype(jnp.int32)
    targets = targets.astype(jnp.int32)
    src_pad = jnp.pad(sources, ((0, 0), (0, SP - SEQ)))

    tgt_idx = jnp.where(targets >= V, UNK, targets).T.reshape(NW, IDX_PW // 128, 128)
    emb_pad = jnp.pad(emb.astype(f32), ((0, 0), (0, EP - E)))
    temb_t = _emb_gather(emb_pad, tgt_idx).reshape(T, B, EP)

    out_base, pc = _tc_forward(
        src_pad, targets.T.reshape(T, B, 1), temb_t, encoded_sources,
        Ws_w.T, Ws_b.reshape(1, H), Wc_w.T, Wc_b.reshape(1, H),
        Wo_w.T, Wo_b.reshape(1, V), W_ih.T, b_ih.reshape(1, 3 * H),
        W_hh.T, b_hh.reshape(1, 3 * H))

    return _sc_scatter(out_base, src_pad, pc)
